# baseline (speedup 1.0000x reference)
import functools

import jax
import jax.numpy as jnp
from jax import lax
from jax.experimental import pallas as pl
from jax.experimental.pallas import tpu as pltpu
from jax.experimental.pallas import tpu_sc as plsc

_NUM_ITEMS = 1000000
_EMBED_DIM = 64
_BATCH = 4096
_HIST = 20

_NC, _NS = 2, 16
_NW = _NC * _NS
_TOTAL = _BATCH * _HIST
_B_PER_W = _TOTAL // _NW
_CHUNK = 128
_N_CHUNKS = _B_PER_W // _CHUNK

_mesh = plsc.VectorSubcoreMesh(core_axis_name="c", subcore_axis_name="s")


@functools.partial(
    pl.kernel,
    mesh=_mesh,
    compiler_params=pltpu.CompilerParams(use_tc_tiling_on_sc=False),
    out_type=jax.ShapeDtypeStruct((_TOTAL, _EMBED_DIM), jnp.float32),
    scratch_types=[
        pltpu.VMEM((_N_CHUNKS, _CHUNK), jnp.int32),
        pltpu.VMEM((_CHUNK, _EMBED_DIM), jnp.float32),
        pltpu.SemaphoreType.DMA,
    ],
)
def _gather_rows(table_hbm, ids_hbm, out_hbm, idx_v, rows_v, sem):
    wid = lax.axis_index("s") * _NC + lax.axis_index("c")
    base = wid * _B_PER_W
    pltpu.sync_copy(ids_hbm.at[wid], idx_v)

    def body(j, carry):
        pltpu.async_copy(table_hbm.at[idx_v.at[j]], rows_v, sem).wait()
        pltpu.sync_copy(rows_v, out_hbm.at[pl.ds(base + j * _CHUNK, _CHUNK)])
        return carry

    lax.fori_loop(0, _N_CHUNKS, body, 0)


def kernel(item_ids, table):
    ids = item_ids.reshape(_NW, _N_CHUNKS, _CHUNK).astype(jnp.int32)
    out = _gather_rows(table, ids)
    return out.reshape(_BATCH, _HIST, _EMBED_DIM)


# speedup vs baseline: 1.0167x; 1.0167x over previous
import functools

import jax
import jax.numpy as jnp
from jax import lax
from jax.experimental import pallas as pl
from jax.experimental.pallas import tpu as pltpu
from jax.experimental.pallas import tpu_sc as plsc

_NUM_ITEMS = 1000000
_EMBED_DIM = 64
_BATCH = 4096
_HIST = 20

_NC, _NS = 2, 16
_NW = _NC * _NS
_TOTAL = _BATCH * _HIST
_B_PER_W = _TOTAL // _NW
_CHUNK = 128
_N_CHUNKS = _B_PER_W // _CHUNK
_G = 5
_S = _N_CHUNKS // _G
_NBUF = 3
_STAGE_ROWS = _G * _CHUNK

_mesh = plsc.VectorSubcoreMesh(core_axis_name="c", subcore_axis_name="s")


@functools.partial(
    pl.kernel,
    mesh=_mesh,
    compiler_params=pltpu.CompilerParams(use_tc_tiling_on_sc=False),
    out_type=jax.ShapeDtypeStruct((_TOTAL, _EMBED_DIM), jnp.float32),
    scratch_types=[
        pltpu.VMEM((_N_CHUNKS, _CHUNK), jnp.int32),
        pltpu.VMEM((_NBUF, _STAGE_ROWS, _EMBED_DIM), jnp.float32),
        pltpu.SemaphoreType.DMA,
        pltpu.SemaphoreType.DMA,
    ],
)
def _gather_rows(table_hbm, ids_hbm, out_hbm, idx_v, rows_v, gsem, wsem):
    wid = lax.axis_index("s") * _NC + lax.axis_index("c")
    base = wid * _B_PER_W
    pltpu.sync_copy(ids_hbm.at[wid], idx_v)

    gathers = [[None] * _G for _ in range(_S)]
    writes = [None] * _S

    def fire(g):
        b = g % _NBUF
        for i in range(_G):
            j = g * _G + i
            gathers[g][i] = pltpu.async_copy(
                table_hbm.at[idx_v.at[j]],
                rows_v.at[b].at[pl.ds(i * _CHUNK, _CHUNK)],
                gsem,
            )

    for g in range(min(_NBUF, _S)):
        fire(g)
    for g in range(_S):
        for d in gathers[g]:
            d.wait()
        writes[g] = pltpu.async_copy(
            rows_v.at[g % _NBUF],
            out_hbm.at[pl.ds(base + g * _STAGE_ROWS, _STAGE_ROWS)],
            wsem,
        )
        nxt = g + _NBUF
        if nxt < _S:
            writes[nxt - _NBUF].wait()
            fire(nxt)
    for g in range(_S):
        if writes[g] is not None and (g + _NBUF >= _S):
            writes[g].wait()


def kernel(item_ids, table):
    ids = item_ids.reshape(_NW, _N_CHUNKS, _CHUNK).astype(jnp.int32)
    out = _gather_rows(table, ids)
    return out.reshape(_BATCH, _HIST, _EMBED_DIM)


# speedup vs baseline: 1.0690x; 1.0514x over previous
import functools

import jax
import jax.numpy as jnp
from jax import lax
from jax.experimental import pallas as pl
from jax.experimental.pallas import tpu as pltpu
from jax.experimental.pallas import tpu_sc as plsc

_NUM_ITEMS = 1000000
_EMBED_DIM = 64
_BATCH = 4096
_HIST = 20

_NC, _NS = 2, 16
_NW = _NC * _NS
_TOTAL = _BATCH * _HIST
_B_PER_W = _TOTAL // _NW
_C = 128
_NCH = _B_PER_W // _C

_mesh = plsc.VectorSubcoreMesh(core_axis_name="c", subcore_axis_name="s")


@functools.partial(
    pl.kernel,
    mesh=_mesh,
    compiler_params=pltpu.CompilerParams(use_tc_tiling_on_sc=True),
    out_type=jax.ShapeDtypeStruct((_TOTAL * _EMBED_DIM,), jnp.float32),
    scratch_types=[
        pltpu.VMEM((_B_PER_W,), jnp.int32),
        pltpu.VMEM((2, _C, 2 * _EMBED_DIM), jnp.float32),
        pltpu.VMEM((2, _C * _EMBED_DIM), jnp.float32),
        pltpu.SemaphoreType.DMA,
        pltpu.SemaphoreType.DMA,
    ],
)
def _gather_rows(table_hbm, ids_hbm, out_hbm, ids_v,
                 rows_v, stage_v, gsem, wsem):
    wid = lax.axis_index("s") * _NC + lax.axis_index("c")
    base = wid * _B_PER_W
    pltpu.sync_copy(ids_hbm.at[wid], ids_v)

    def fire(j):
        b = j % 2
        return pltpu.async_copy(
            table_hbm.at[ids_v.at[pl.ds(j * _C, _C)]], rows_v.at[b], gsem)

    gathers = [None] * _NCH
    writes = [None] * _NCH
    gathers[0] = fire(0)
    for j in range(_NCH):
        b = j % 2
        if j + 1 < _NCH:
            gathers[j + 1] = fire(j + 1)
        gathers[j].wait()
        if j >= 2:
            writes[j - 2].wait()

        def extract(k, carry):
            for q in range(_EMBED_DIM // 16):
                stage_v[b, pl.ds(k * _EMBED_DIM + q * 16, 16)] = (
                    rows_v[b, k, pl.ds(q * 16, 16)])
            return carry

        lax.fori_loop(0, _C, extract, 0)
        writes[j] = pltpu.async_copy(
            stage_v.at[b],
            out_hbm.at[pl.ds((base + j * _C) * _EMBED_DIM, _C * _EMBED_DIM)],
            wsem,
        )
    writes[_NCH - 2].wait()
    writes[_NCH - 1].wait()


def kernel(item_ids, table):
    ids = item_ids.reshape(_NW, _B_PER_W).astype(jnp.int32)
    tpad = jnp.pad(table, ((0, 0), (0, _EMBED_DIM)))
    out = _gather_rows(tpad, ids)
    return out.reshape(_BATCH, _HIST, _EMBED_DIM)
